# baseline (speedup 1.0000x reference)
import functools

import jax
import jax.numpy as jnp
from jax import lax
from jax.experimental import pallas as pl
from jax.experimental.pallas import tpu as pltpu
from jax.experimental.pallas import tpu_sc as plsc

B = 16384
EMB = 32
HID = 128
OUT = 64
NB = 16
BB = B // NB


def _make_sc_gather():
    info = plsc.get_sparse_core_info()
    nc, ns = info.num_cores, info.num_subcores
    nw = nc * ns
    bpw = B // nw
    CH = 64
    nch = bpw // CH
    mesh = plsc.VectorSubcoreMesh(core_axis_name="c", subcore_axis_name="s")

    @functools.partial(
        pl.kernel,
        mesh=mesh,
        out_type=jax.ShapeDtypeStruct((B, EMB), jnp.float32),
        scratch_types=[
            pltpu.VMEM((4, 128), jnp.int32),
            pltpu.VMEM((bpw, EMB), jnp.float32),
            pltpu.SemaphoreType.DMA,
        ],
    )
    def sc_gather(table_hbm, idx_hbm, out_hbm, idx_v, rows_v, sem):
        wid = lax.axis_index("s") * nc + lax.axis_index("c")
        pltpu.sync_copy(idx_hbm.at[pl.ds(wid * 4, 4)], idx_v)
        for r in range(4):
            def issue(c16, _, r=r):
                v16 = idx_v[r, pl.ds(c16 * 16, 16)]
                for l in range(16):
                    idx_s = v16[l]
                    tid = lax.shift_right_logical(idx_s, 3)
                    rid = lax.bitwise_and(idx_s, 7)
                    g = r * 128 + c16 * 16 + l
                    pltpu.async_copy(table_hbm.at[tid, rid], rows_v.at[g],
                                     sem)
                return 0

            lax.fori_loop(0, 8, issue, 0)

        def drain(i, _):
            pltpu.make_async_copy(table_hbm.at[0, 0], rows_v.at[0],
                                  sem).wait()
            return 0

        lax.fori_loop(0, bpw, drain, 0)
        pltpu.sync_copy(rows_v, out_hbm.at[pl.ds(wid * bpw, bpw)])

    return sc_gather


def _tc_onehot_body(g_ref, a_ref, o_ref, e_ref, w1t_ref, h0_ref, proj_scr):
    j = pl.program_id(0)

    @pl.when(j == 0)
    def _():
        proj_scr[:] = jnp.dot(e_ref[:], w1t_ref[:],
                              preferred_element_type=jnp.float32)

    cols = lax.broadcasted_iota(jnp.int32, (BB, HID), 1)
    g = g_ref[0, 0, :][:, None]
    a = a_ref[0, 0, :][:, None]
    o = o_ref[0, 0, :][:, None]
    oh = ((cols == g) | (cols == a) | (cols == o)).astype(jnp.float32)
    h0_ref[:] = jnp.dot(oh, proj_scr[:], preferred_element_type=jnp.float32)


def _tc_onehot(g3, a3, o3, e_mat, w1t):
    return pl.pallas_call(
        _tc_onehot_body,
        grid=(NB,),
        in_specs=[
            pl.BlockSpec((1, 1, BB), lambda j: (j, 0, 0)),
            pl.BlockSpec((1, 1, BB), lambda j: (j, 0, 0)),
            pl.BlockSpec((1, 1, BB), lambda j: (j, 0, 0)),
            pl.BlockSpec((HID, HID), lambda j: (0, 0)),
            pl.BlockSpec((HID, HID), lambda j: (0, 0)),
        ],
        out_specs=pl.BlockSpec((BB, HID), lambda j: (j, 0)),
        out_shape=jax.ShapeDtypeStruct((B, HID), jnp.float32),
        scratch_shapes=[pltpu.VMEM((HID, HID), jnp.float32)],
    )(g3, a3, o3, e_mat, w1t)


def _tc_fused_body(ue_ref, h0_ref, w1tu_ref,
                   gamma_ref, beta_ref, w2_ref, b2_ref,
                   ot_ref, h_scr, stats_scr):
    p = pl.program_id(0)
    j = pl.program_id(1)

    @pl.when(jnp.logical_and(p == 0, j == 0))
    def _():
        stats_scr[:] = jnp.zeros_like(stats_scr)

    @pl.when(p == 0)
    def _():
        h = (h0_ref[:]
             + jnp.dot(ue_ref[:], w1tu_ref[:],
                       preferred_element_type=jnp.float32))
        h_scr[pl.ds(j * BB, BB), :] = h
        s0 = jnp.sum(h, axis=0, keepdims=True)
        s1 = jnp.sum(h * h, axis=0, keepdims=True)
        upd = jnp.concatenate([s0, s1, jnp.zeros((6, HID), jnp.float32)],
                              axis=0)
        stats_scr[:] = stats_scr[:] + upd

    @pl.when(p == 1)
    def _():
        stats = stats_scr[:]
        mean = stats[0:1, :] * (1.0 / B)
        var = stats[1:2, :] * (1.0 / B) - mean * mean
        scale = lax.rsqrt(var + 1e-5) * gamma_ref[:]
        h = h_scr[pl.ds(j * BB, BB), :]
        hn = jnp.maximum((h - mean) * scale + beta_ref[:], 0.0)
        ot = lax.dot_general(w2_ref[:], hn, (((1,), (1,)), ((), ())),
                             preferred_element_type=jnp.float32)
        ot = ot + b2_ref[:]
        n2 = jnp.sum(ot * ot, axis=0, keepdims=True)
        ot_ref[:] = ot * lax.rsqrt(jnp.maximum(n2, 1e-24))


def _tc_fused(ue, h0, w1tu, gamma, beta, w2, b2):
    return pl.pallas_call(
        _tc_fused_body,
        grid=(2, NB),
        in_specs=[
            pl.BlockSpec((BB, EMB), lambda p, j: ((1 - p) * j, 0)),
            pl.BlockSpec((BB, HID), lambda p, j: ((1 - p) * j, 0)),
            pl.BlockSpec((EMB, HID), lambda p, j: (0, 0)),
            pl.BlockSpec((1, HID), lambda p, j: (0, 0)),
            pl.BlockSpec((1, HID), lambda p, j: (0, 0)),
            pl.BlockSpec((OUT, HID), lambda p, j: (0, 0)),
            pl.BlockSpec((OUT, 1), lambda p, j: (0, 0)),
        ],
        out_specs=pl.BlockSpec((OUT, BB), lambda p, j: (0, p * j)),
        out_shape=jax.ShapeDtypeStruct((OUT, B), jnp.float32),
        scratch_shapes=[
            pltpu.VMEM((B, HID), jnp.float32),
            pltpu.VMEM((8, HID), jnp.float32),
        ],
    )(ue, h0, w1tu, gamma, beta, w2, b2)


def _small_tables_matrix(gender_table, age_table, occupation_table):
    z = jnp.zeros
    r0 = jnp.concatenate([z((4, EMB)), gender_table, z((4, 2 * EMB))], axis=1)
    r1 = jnp.concatenate([z((8, 2 * EMB)), age_table, z((8, EMB))], axis=1)
    r2 = jnp.concatenate([z((32, 3 * EMB)), occupation_table], axis=1)
    r3 = z((HID - 44, HID))
    return jnp.concatenate([r0, r1, r2, r3], axis=0)


def kernel(user_idx, gender_idx, age_idx, occupation_idx,
           user_table, gender_table, age_table, occupation_table,
           W1, b1, gamma, beta, W2, b2):
    del b1
    ut3 = user_table.reshape(user_table.shape[0] // 8, 8, EMB)
    ue = _make_sc_gather()(ut3,
                           user_idx.astype(jnp.int32).reshape(B // 128, 128))
    g3 = gender_idx.astype(jnp.int32).reshape(NB, 1, BB)
    a3 = (age_idx.astype(jnp.int32) + 4).reshape(NB, 1, BB)
    o3 = (occupation_idx.astype(jnp.int32) + 12).reshape(NB, 1, BB)
    e_mat = _small_tables_matrix(gender_table, age_table, occupation_table)
    w1t = W1.T
    w1tu = w1t[:EMB]
    h0 = _tc_onehot(g3, a3, o3, e_mat, w1t)
    ot = _tc_fused(ue, h0, w1tu,
                   gamma.reshape(1, HID), beta.reshape(1, HID),
                   W2, b2.reshape(OUT, 1))
    return ot.T


# speedup vs baseline: 1.0310x; 1.0310x over previous
import functools

import jax
import jax.numpy as jnp
from jax import lax
from jax.experimental import pallas as pl
from jax.experimental.pallas import tpu as pltpu
from jax.experimental.pallas import tpu_sc as plsc

B = 16384
EMB = 32
HID = 128
OUT = 64
NB = 16
BB = B // NB


def _make_sc_gather():
    info = plsc.get_sparse_core_info()
    nc, ns = info.num_cores, info.num_subcores
    nw = nc * ns
    bpw = B // nw
    mesh = plsc.VectorSubcoreMesh(core_axis_name="c", subcore_axis_name="s")

    @functools.partial(
        pl.kernel,
        mesh=mesh,
        out_type=jax.ShapeDtypeStruct((B, EMB), jnp.float32),
        scratch_types=[
            pltpu.VMEM((4, 128), jnp.int32),
            pltpu.VMEM((bpw, EMB), jnp.float32),
            pltpu.SemaphoreType.DMA,
        ],
    )
    def sc_gather(table_hbm, idx_hbm, out_hbm, idx_v, rows_v, sem):
        wid = lax.axis_index("s") * nc + lax.axis_index("c")
        pltpu.sync_copy(idx_hbm.at[pl.ds(wid * 4, 4)], idx_v)
        for r in range(4):
            def issue(c16, _, r=r):
                v16 = idx_v[r, pl.ds(c16 * 16, 16)]
                for l in range(16):
                    idx_s = v16[l]
                    tid = lax.shift_right_logical(idx_s, 3)
                    rid = lax.bitwise_and(idx_s, 7)
                    g = r * 128 + c16 * 16 + l
                    pltpu.async_copy(table_hbm.at[tid, rid], rows_v.at[g],
                                     sem)
                return 0

            lax.fori_loop(0, 8, issue, 0)

        def drain(i, _):
            pltpu.make_async_copy(table_hbm.at[0, 0], rows_v.at[0],
                                  sem).wait()
            return 0

        lax.fori_loop(0, bpw, drain, 0)
        pltpu.sync_copy(rows_v, out_hbm.at[pl.ds(wid * bpw, bpw)])

    return sc_gather


def _tc_fused_body(ue_ref, g_ref, a_ref, o_ref, e_ref, w1t_ref, w1tu_ref,
                   gamma_ref, beta_ref, w2_ref, b2_ref,
                   ot_ref, h_scr, stats_scr, proj_scr):
    p = pl.program_id(0)
    j = pl.program_id(1)

    @pl.when(jnp.logical_and(p == 0, j == 0))
    def _():
        proj_scr[:] = jnp.dot(e_ref[:], w1t_ref[:],
                              preferred_element_type=jnp.float32)
        stats_scr[:] = jnp.zeros_like(stats_scr)

    @pl.when(p == 0)
    def _():
        cols = lax.broadcasted_iota(jnp.int32, (BB, HID), 1)
        g = g_ref[0, 0, :][:, None]
        a = a_ref[0, 0, :][:, None]
        o = o_ref[0, 0, :][:, None]
        oh = ((cols == g) | (cols == a) | (cols == o)).astype(jnp.float32)
        h = (jnp.dot(oh, proj_scr[:], preferred_element_type=jnp.float32)
             + jnp.dot(ue_ref[:], w1tu_ref[:],
                       preferred_element_type=jnp.float32))
        h_scr[pl.ds(j * BB, BB), :] = h
        s0 = jnp.sum(h, axis=0, keepdims=True)
        s1 = jnp.sum(h * h, axis=0, keepdims=True)
        upd = jnp.concatenate([s0, s1, jnp.zeros((6, HID), jnp.float32)],
                              axis=0)
        stats_scr[:] = stats_scr[:] + upd

    @pl.when(p == 1)
    def _():
        stats = stats_scr[:]
        mean = stats[0:1, :] * (1.0 / B)
        var = stats[1:2, :] * (1.0 / B) - mean * mean
        scale = lax.rsqrt(var + 1e-5) * gamma_ref[:]
        h = h_scr[pl.ds(j * BB, BB), :]
        hn = jnp.maximum((h - mean) * scale + beta_ref[:], 0.0)
        ot = lax.dot_general(w2_ref[:], hn, (((1,), (1,)), ((), ())),
                             preferred_element_type=jnp.float32)
        ot = ot + b2_ref[:]
        n2 = jnp.sum(ot * ot, axis=0, keepdims=True)
        ot_ref[:] = ot * lax.rsqrt(jnp.maximum(n2, 1e-24))


def _tc_fused(ue, g3, a3, o3, e_mat, w1t, w1tu, gamma, beta, w2, b2):
    return pl.pallas_call(
        _tc_fused_body,
        grid=(2, NB),
        in_specs=[
            pl.BlockSpec((BB, EMB), lambda p, j: ((1 - p) * j, 0)),
            pl.BlockSpec((1, 1, BB), lambda p, j: ((1 - p) * j, 0, 0)),
            pl.BlockSpec((1, 1, BB), lambda p, j: ((1 - p) * j, 0, 0)),
            pl.BlockSpec((1, 1, BB), lambda p, j: ((1 - p) * j, 0, 0)),
            pl.BlockSpec((HID, HID), lambda p, j: (0, 0)),
            pl.BlockSpec((HID, HID), lambda p, j: (0, 0)),
            pl.BlockSpec((EMB, HID), lambda p, j: (0, 0)),
            pl.BlockSpec((1, HID), lambda p, j: (0, 0)),
            pl.BlockSpec((1, HID), lambda p, j: (0, 0)),
            pl.BlockSpec((OUT, HID), lambda p, j: (0, 0)),
            pl.BlockSpec((OUT, 1), lambda p, j: (0, 0)),
        ],
        out_specs=pl.BlockSpec((OUT, BB), lambda p, j: (0, p * j)),
        out_shape=jax.ShapeDtypeStruct((OUT, B), jnp.float32),
        scratch_shapes=[
            pltpu.VMEM((B, HID), jnp.float32),
            pltpu.VMEM((8, HID), jnp.float32),
            pltpu.VMEM((HID, HID), jnp.float32),
        ],
    )(ue, g3, a3, o3, e_mat, w1t, w1tu, gamma, beta, w2, b2)


def _small_tables_matrix(gender_table, age_table, occupation_table):
    z = jnp.zeros
    r0 = jnp.concatenate([z((4, EMB)), gender_table, z((4, 2 * EMB))], axis=1)
    r1 = jnp.concatenate([z((8, 2 * EMB)), age_table, z((8, EMB))], axis=1)
    r2 = jnp.concatenate([z((32, 3 * EMB)), occupation_table], axis=1)
    r3 = z((HID - 44, HID))
    return jnp.concatenate([r0, r1, r2, r3], axis=0)


def kernel(user_idx, gender_idx, age_idx, occupation_idx,
           user_table, gender_table, age_table, occupation_table,
           W1, b1, gamma, beta, W2, b2):
    del b1
    ut3 = user_table.reshape(user_table.shape[0] // 8, 8, EMB)
    ue = _make_sc_gather()(ut3,
                           user_idx.astype(jnp.int32).reshape(B // 128, 128))
    g3 = gender_idx.astype(jnp.int32).reshape(NB, 1, BB)
    a3 = (age_idx.astype(jnp.int32) + 4).reshape(NB, 1, BB)
    o3 = (occupation_idx.astype(jnp.int32) + 12).reshape(NB, 1, BB)
    e_mat = _small_tables_matrix(gender_table, age_table, occupation_table)
    w1t = W1.T
    w1tu = w1t[:EMB]
    ot = _tc_fused(ue, g3, a3, o3, e_mat, w1t, w1tu,
                   gamma.reshape(1, HID), beta.reshape(1, HID),
                   W2, b2.reshape(OUT, 1))
    return ot.T
